# Initial kernel scaffold; baseline (speedup 1.0000x reference)
import jax
import jax.numpy as jnp
from jax.experimental import pallas as pl


def kernel(step, lprobs, scores):
    raise NotImplementedError("write your pallas kernel here")


# speedup vs baseline: 7.6506x; 7.6506x over previous
import functools

import jax
import jax.numpy as jnp
from jax import lax
from jax.experimental import pallas as pl
from jax.experimental.pallas import tpu as pltpu
from jax.experimental.pallas import tpu_sc as plsc

BSZ = 64
NBEAMS = 8
VOCAB = 100000
K = 16
ROW = NBEAMS * VOCAB
SHARD = 50000
NSHARDS = ROW // SHARD
LANES = 16
VREGS = SHARD // LANES
GROUP = 25
NGROUPS = VREGS // GROUP
NWORKERS = 32
ROWS_PER_W = BSZ // NWORKERS

NEG_INF = float("-inf")


def _neg_inf_vec():
    return jnp.full((LANES,), NEG_INF, jnp.float32)


def _topk_body(lp_hbm, bias_hbm, out_s, out_i, out_b,
               buf0, buf1, biasv, stv, sti, stb, sem0, sem1):
    cid = lax.axis_index("c")
    sid = lax.axis_index("s")
    wid = sid * 2 + cid

    pltpu.sync_copy(bias_hbm.at[pl.ds(wid * LANES, LANES)], biasv)
    bias_all = biasv[...]
    lane = lax.broadcasted_iota(jnp.int32, (LANES,), 0)

    zeros_idx = jnp.zeros((LANES,), jnp.int32)

    def t_safe_of(cur_val, bias_b):
        t = jnp.take_along_axis(cur_val, zeros_idx, axis=0)
        ta = t - bias_b
        margin = (jnp.abs(t) + jnp.abs(bias_b) + 8.0) * 1e-6
        return ta - margin

    def process(buf, s_idx, rl, cur):
        cur_val, cur_idx = cur
        beam = s_idx // 2
        bsel = jnp.full((LANES,), rl * NBEAMS, jnp.int32) + beam
        bias_b = jnp.take_along_axis(bias_all, bsel, axis=0)
        flat_base = s_idx * SHARD

        def group(g, carry):
            cv, ci, ts = carry
            base = g * (GROUP * LANES)
            vs = [buf[pl.ds(base + LANES * j, LANES)] for j in range(GROUP)]
            gmax = functools.reduce(jnp.maximum, vs)
            hit = jnp.any(gmax > ts)

            def slow(c):
                cv_in, ci_in, ts_in = c

                def inner(j, cc):
                    cv2, ci2 = cc
                    v = buf[pl.ds(base + LANES * j, LANES)]
                    h = jnp.any(v > ts_in)

                    def merge(cc2):
                        cv3, ci3 = cc2
                        bv = v + bias_b
                        fi = flat_base + base + LANES * j + lane
                        sk, si = plsc.sort_key_val(bv, fi)
                        rk = lax.rev(sk, (0,))
                        ri = lax.rev(si, (0,))
                        ge = cv3 >= rk
                        hv = jnp.where(ge, cv3, rk)
                        hi = jnp.where(ge, ci3, ri)
                        nv, ni = plsc.sort_key_val(hv, hi)
                        return nv, ni

                    return lax.cond(h, merge, lambda x: x, (cv2, ci2))

                cv_o, ci_o = lax.fori_loop(0, GROUP, inner, (cv_in, ci_in))
                return cv_o, ci_o, t_safe_of(cv_o, bias_b)

            return lax.cond(hit, slow, lambda c: c, (cv, ci, ts))

        cv, ci, _ = lax.fori_loop(
            0, NGROUPS, group,
            (cur_val, cur_idx, t_safe_of(cur_val, bias_b)))
        return cv, ci

    for rl in range(ROWS_PER_W):
        row = wid * ROWS_PER_W + rl
        row_base = row * ROW

        def start(s, buf, sem):
            off = pl.multiple_of(row_base + s * SHARD, 8)
            pltpu.async_copy(lp_hbm.at[pl.ds(off, SHARD)], buf, sem)

        def waitfor(s, buf, sem):
            off = pl.multiple_of(row_base + s * SHARD, 8)
            pltpu.make_async_copy(lp_hbm.at[pl.ds(off, SHARD)], buf, sem).wait()

        start(0, buf0, sem0)
        init = (_neg_inf_vec(), jnp.zeros((LANES,), jnp.int32))

        def it(i, cur, rl=rl):
            s = 2 * i
            start(s + 1, buf1, sem1)
            waitfor(s, buf0, sem0)
            cur = process(buf0, s, rl, cur)

            @pl.when(s + 2 < NSHARDS)
            def _():
                start(s + 2, buf0, sem0)

            waitfor(s + 1, buf1, sem1)
            cur = process(buf1, s + 1, rl, cur)
            return cur

        cv, ci = lax.fori_loop(0, NSHARDS // 2, it, init)

        vd = lax.rev(cv, (0,))
        idd = lax.rev(ci, (0,))
        stv[...] = vd
        sti[...] = idd % VOCAB
        stb[...] = idd // VOCAB
        pltpu.sync_copy(stv, out_s.at[row])
        pltpu.sync_copy(sti, out_i.at[row])
        pltpu.sync_copy(stb, out_b.at[row])


_topk_call = functools.partial(
    pl.kernel,
    out_type=(
        jax.ShapeDtypeStruct((BSZ, K), jnp.float32),
        jax.ShapeDtypeStruct((BSZ, K), jnp.int32),
        jax.ShapeDtypeStruct((BSZ, K), jnp.int32),
    ),
    mesh=plsc.VectorSubcoreMesh(core_axis_name="c", subcore_axis_name="s"),
    scratch_types=[
        pltpu.VMEM((SHARD,), jnp.float32),
        pltpu.VMEM((SHARD,), jnp.float32),
        pltpu.VMEM((LANES,), jnp.float32),
        pltpu.VMEM((LANES,), jnp.float32),
        pltpu.VMEM((LANES,), jnp.int32),
        pltpu.VMEM((LANES,), jnp.int32),
        pltpu.SemaphoreType.DMA,
        pltpu.SemaphoreType.DMA,
    ],
    compiler_params=pltpu.CompilerParams(needs_layout_passes=False),
)(_topk_body)


def kernel(step, lprobs, scores):
    step = jnp.asarray(step)
    col = lax.dynamic_index_in_dim(
        scores, jnp.maximum(step - 1, 0), axis=2, keepdims=False)
    step0_bias = jnp.where(jnp.arange(NBEAMS) == 0, 0.0, NEG_INF)[None, :]
    bias = jnp.where(step == 0, step0_bias, col).astype(jnp.float32)
    return _topk_call(lprobs.reshape(-1), bias.reshape(-1))


# speedup vs baseline: 17.6751x; 2.3103x over previous
import functools

import jax
import jax.numpy as jnp
from jax import lax
from jax.experimental import pallas as pl
from jax.experimental.pallas import tpu as pltpu
from jax.experimental.pallas import tpu_sc as plsc

BSZ = 64
NBEAMS = 8
VOCAB = 100000
K = 16
LANES = 16

TILE = 128
NTILES_FULL = VOCAB // TILE
VOCAB_MAIN = NTILES_FULL * TILE
VOCAB_TAIL = VOCAB - VOCAB_MAIN

SH_TILES = 48
SH_W = SH_TILES * TILE
NSH = 16
T13_TILES = NTILES_FULL - NSH * SH_TILES
T13_W = T13_TILES * TILE
T13_BASE = NSH * SH_W

GROUP = 48
NGROUPS = (SH_W // LANES) // GROUP
G13 = 52
NG13 = (T13_W // LANES) // G13

NWORKERS = 32
ROWS_PER_W = BSZ // NWORKERS

NEG_INF = float("-inf")


def _neg_inf_vec():
    return jnp.full((LANES,), NEG_INF, jnp.float32)


def _tree_max(vs):
    vs = list(vs)
    while len(vs) > 1:
        vs = [jnp.maximum(a, b) for a, b in zip(vs[::2], vs[1::2])] + (
            [vs[-1]] if len(vs) % 2 else [])
    return vs[0]


def _topk_body(lp_hbm, tail_hbm, bias_hbm, out_s, out_i, out_b,
               buf0, buf1, buf13, tbuf, biasv, stv, sti, stb,
               sem0, sem1, sem13):
    cid = lax.axis_index("c")
    sid = lax.axis_index("s")
    wid = sid * 2 + cid

    pltpu.sync_copy(bias_hbm.at[pl.ds(wid * LANES, LANES)], biasv)
    bias_all = biasv[...]
    lane = lax.broadcasted_iota(jnp.int32, (LANES,), 0)
    zeros_idx = jnp.zeros((LANES,), jnp.int32)

    def t_safe_of(cur_val, bias_b):
        t = jnp.take_along_axis(cur_val, zeros_idx, axis=0)
        ta = t - bias_b
        margin = (jnp.abs(t) + jnp.abs(bias_b) + 8.0) * 1e-6
        return ta - margin

    def merge_vec(cv, ci, bv, fi):
        sk, si = plsc.sort_key_val(bv, fi)
        rk = lax.rev(sk, (0,))
        ri = lax.rev(si, (0,))
        ge = cv >= rk
        hv = jnp.where(ge, cv, rk)
        hi = jnp.where(ge, ci, ri)
        nv, ni = plsc.sort_key_val(hv, hi)
        return nv, ni

    def mk_proc(buf, ngroups, group, vbase, rl):
        def beam_fn(b, cur):
            cv0, ci0 = cur
            bsel = jnp.full((LANES,), rl * NBEAMS, jnp.int32) + b
            bias_b = jnp.take_along_axis(bias_all, bsel, axis=0)
            fbase = b * VOCAB + vbase

            def grp(g, carry):
                cv, ci, ts = carry
                base = g * (group * LANES)
                vs = [buf[b, pl.ds(base + LANES * j, LANES)]
                      for j in range(group)]
                gmax = _tree_max(vs)
                hit = jnp.any(gmax > ts)

                def slow(c):
                    cv_in, ci_in, ts_in = c

                    def inner(j, cc):
                        cv2, ci2 = cc
                        v = buf[b, pl.ds(base + LANES * j, LANES)]
                        h = jnp.any(v > ts_in)

                        def do(cc2):
                            cv3, ci3 = cc2
                            bv = v + bias_b
                            fi = fbase + base + LANES * j + lane
                            return merge_vec(cv3, ci3, bv, fi)

                        return lax.cond(h, do, lambda x: x, (cv2, ci2))

                    cv_o, ci_o = lax.fori_loop(0, group, inner, (cv_in, ci_in))
                    return cv_o, ci_o, t_safe_of(cv_o, bias_b)

                return lax.cond(hit, slow, lambda c: c, (cv, ci, ts))

            cv1, ci1, _ = lax.fori_loop(
                0, ngroups, grp, (cv0, ci0, t_safe_of(cv0, bias_b)))
            return cv1, ci1

        return beam_fn

    def row_fn(rl, _):
        row = wid * ROWS_PER_W + rl

        def start(k, buf, sem):
            off = pl.multiple_of(k * SH_W, TILE)
            pltpu.async_copy(lp_hbm.at[row, :, pl.ds(off, SH_W)], buf, sem)

        def waitfor(k, buf, sem):
            off = pl.multiple_of(k * SH_W, TILE)
            pltpu.make_async_copy(
                lp_hbm.at[row, :, pl.ds(off, SH_W)], buf, sem).wait()

        def proc(buf, k, cur):
            vbase = k * SH_W
            return lax.fori_loop(
                0, NBEAMS, mk_proc(buf, NGROUPS, GROUP, vbase, rl), cur)

        start(0, buf0, sem0)
        init = (_neg_inf_vec(), jnp.zeros((LANES,), jnp.int32))

        def it(i, cur):
            s = 2 * i
            start(s + 1, buf1, sem1)
            waitfor(s, buf0, sem0)
            cur = proc(buf0, s, cur)

            @pl.when(s + 2 < NSH)
            def _():
                start(s + 2, buf0, sem0)

            @pl.when(s + 2 >= NSH)
            def _():
                pltpu.async_copy(
                    lp_hbm.at[row, :, pl.ds(T13_BASE, T13_W)], buf13, sem13)

            waitfor(s + 1, buf1, sem1)
            cur = proc(buf1, s + 1, cur)
            return cur

        cur = lax.fori_loop(0, NSH // 2, it, init)

        pltpu.make_async_copy(
            lp_hbm.at[row, :, pl.ds(T13_BASE, T13_W)], buf13, sem13).wait()
        cur = lax.fori_loop(
            0, NBEAMS, mk_proc(buf13, NG13, G13, T13_BASE, rl), cur)

        pltpu.sync_copy(
            tail_hbm.at[pl.ds(row * (NBEAMS * VOCAB_TAIL),
                              NBEAMS * VOCAB_TAIL)], tbuf)

        def tail_fn(m, cc):
            cv2, ci2 = cc
            b = m // 2
            v = tbuf[pl.ds(m * LANES, LANES)]
            bsel = jnp.full((LANES,), rl * NBEAMS, jnp.int32) + b
            bias_b = jnp.take_along_axis(bias_all, bsel, axis=0)
            ts = t_safe_of(cv2, bias_b)
            h = jnp.any(v > ts)

            def do(cc2):
                cv3, ci3 = cc2
                bv = v + bias_b
                fi = b * VOCAB + VOCAB_MAIN + (m % 2) * LANES + lane
                return merge_vec(cv3, ci3, bv, fi)

            return lax.cond(h, do, lambda x: x, (cv2, ci2))

        cv, ci = lax.fori_loop(0, NBEAMS * 2, tail_fn, cur)

        vd = lax.rev(cv, (0,))
        idd = lax.rev(ci, (0,))
        stv[...] = vd
        sti[...] = idd % VOCAB
        stb[...] = idd // VOCAB
        pltpu.sync_copy(stv, out_s.at[row])
        pltpu.sync_copy(sti, out_i.at[row])
        pltpu.sync_copy(stb, out_b.at[row])
        return 0

    lax.fori_loop(0, ROWS_PER_W, row_fn, 0)


_topk_call = functools.partial(
    pl.kernel,
    out_type=(
        jax.ShapeDtypeStruct((BSZ, K), jnp.float32),
        jax.ShapeDtypeStruct((BSZ, K), jnp.int32),
        jax.ShapeDtypeStruct((BSZ, K), jnp.int32),
    ),
    mesh=plsc.VectorSubcoreMesh(core_axis_name="c", subcore_axis_name="s"),
    scratch_types=[
        pltpu.VMEM((NBEAMS, SH_W), jnp.float32),
        pltpu.VMEM((NBEAMS, SH_W), jnp.float32),
        pltpu.VMEM((NBEAMS, T13_W), jnp.float32),
        pltpu.VMEM((NBEAMS * VOCAB_TAIL,), jnp.float32),
        pltpu.VMEM((LANES,), jnp.float32),
        pltpu.VMEM((LANES,), jnp.float32),
        pltpu.VMEM((LANES,), jnp.int32),
        pltpu.VMEM((LANES,), jnp.int32),
        pltpu.SemaphoreType.DMA,
        pltpu.SemaphoreType.DMA,
        pltpu.SemaphoreType.DMA,
    ],
    compiler_params=pltpu.CompilerParams(needs_layout_passes=False),
)(_topk_body)


def kernel(step, lprobs, scores):
    step = jnp.asarray(step)
    col = lax.dynamic_index_in_dim(
        scores, jnp.maximum(step - 1, 0), axis=2, keepdims=False)
    step0_bias = jnp.where(jnp.arange(NBEAMS) == 0, 0.0, NEG_INF)[None, :]
    bias = jnp.where(step == 0, step0_bias, col).astype(jnp.float32)
    tail = lprobs[:, :, VOCAB_MAIN:].reshape(-1)
    return _topk_call(lprobs, tail, bias.reshape(-1))


# speedup vs baseline: 19.6042x; 1.1091x over previous
import functools

import jax
import jax.numpy as jnp
from jax import lax
from jax.experimental import pallas as pl
from jax.experimental.pallas import tpu as pltpu
from jax.experimental.pallas import tpu_sc as plsc

BSZ = 64
NBEAMS = 8
VOCAB = 100000
K = 16
LANES = 16

TILE = 128
NTILES_FULL = VOCAB // TILE
VOCAB_MAIN = NTILES_FULL * TILE
VOCAB_TAIL = VOCAB - VOCAB_MAIN

SH_TILES = 48
SH_W = SH_TILES * TILE
NSH = 16
T13_TILES = NTILES_FULL - NSH * SH_TILES
T13_W = T13_TILES * TILE
T13_BASE = NSH * SH_W

GROUP = 48
NGROUPS = (SH_W // LANES) // GROUP
G13 = 52
NG13 = (T13_W // LANES) // G13

NWORKERS = 32
ROWS_PER_W = BSZ // NWORKERS

NEG_INF = float("-inf")


def _neg_inf_vec():
    return jnp.full((LANES,), NEG_INF, jnp.float32)


def _tree_max(vs):
    vs = list(vs)
    while len(vs) > 1:
        vs = [jnp.maximum(a, b) for a, b in zip(vs[::2], vs[1::2])] + (
            [vs[-1]] if len(vs) % 2 else [])
    return vs[0]


def _topk_body(lp_hbm, tail_hbm, bias_hbm, out_s, out_i, out_b,
               buf0, buf1, buf13, tbuf, biasv, stv, sti, stb,
               sem0, sem1, sem13):
    cid = lax.axis_index("c")
    sid = lax.axis_index("s")
    wid = sid * 2 + cid

    pltpu.sync_copy(bias_hbm.at[pl.ds(wid * LANES, LANES)], biasv)
    bias_all = biasv[...]
    lane = lax.broadcasted_iota(jnp.int32, (LANES,), 0)
    zeros_idx = jnp.zeros((LANES,), jnp.int32)

    def t_safe_of(cur_val, bias_b):
        t = jnp.take_along_axis(cur_val, zeros_idx, axis=0)
        ta = t - bias_b
        margin = (jnp.abs(t) + jnp.abs(bias_b) + 8.0) * 1e-6
        return ta - margin

    def merge_vec(cv, ci, bv, fi):
        sk, si = plsc.sort_key_val(bv, fi)
        rk = lax.rev(sk, (0,))
        ri = lax.rev(si, (0,))
        ge = cv >= rk
        hv = jnp.where(ge, cv, rk)
        hi = jnp.where(ge, ci, ri)
        nv, ni = plsc.sort_key_val(hv, hi)
        return nv, ni

    pos_inf_vec = jnp.full((LANES,), float("inf"), jnp.float32)

    def mk_proc(buf, ngroups, group, vbase, rl):
        def beam_fn(b, cur):
            cv0, ci0 = cur
            bsel = jnp.full((LANES,), rl * NBEAMS, jnp.int32) + b
            bias_b = jnp.take_along_axis(bias_all, bsel, axis=0)
            fbase = b * VOCAB + vbase

            def grp(g, carry):
                cv, ci, ts = carry
                base = g * (group * LANES)
                vs = [buf[b, pl.ds(base + LANES * j, LANES)]
                      for j in range(group)]
                gmax = _tree_max(vs)
                hit = jnp.any(gmax > ts)

                def slow(c):
                    cv_in, ci_in, _ = c

                    def sweep(ceil_v, ceil_j):
                        gm = _neg_inf_vec()
                        gj = jnp.zeros((LANES,), jnp.int32)
                        for j in range(group):
                            bv = buf[b, pl.ds(base + LANES * j, LANES)] + bias_b
                            jj = jnp.full((LANES,), j, jnp.int32)
                            elig = (bv < ceil_v) | (
                                (bv == ceil_v) & (jj > ceil_j))
                            c2 = elig & (bv > gm)
                            gm = jnp.where(c2, bv, gm)
                            gj = jnp.where(c2, jj, gj)
                        return gm, gj

                    gm0, gj0 = sweep(pos_inf_vec,
                                     jnp.full((LANES,), -1, jnp.int32))
                    t0 = jnp.take_along_axis(cv_in, zeros_idx, axis=0)
                    hit0 = jnp.any(gm0 > t0)

                    def w_cond(st):
                        return st[4]

                    def w_body(st):
                        cv2, ci2, gm, gj, _ = st
                        fi = fbase + base + gj * LANES + lane
                        cv2, ci2 = merge_vec(cv2, ci2, gm, fi)
                        gm2, gj2 = sweep(gm, gj)
                        t_ex = jnp.take_along_axis(cv2, zeros_idx, axis=0)
                        return (cv2, ci2, gm2, gj2, jnp.any(gm2 > t_ex))

                    cv_o, ci_o, _, _, _ = lax.while_loop(
                        w_cond, w_body, (cv_in, ci_in, gm0, gj0, hit0))
                    return cv_o, ci_o, t_safe_of(cv_o, bias_b)

                return lax.cond(hit, slow, lambda c: c, (cv, ci, ts))

            cv1, ci1, _ = lax.fori_loop(
                0, ngroups, grp, (cv0, ci0, t_safe_of(cv0, bias_b)))
            return cv1, ci1

        return beam_fn

    def row_fn(rl, _):
        row = wid * ROWS_PER_W + rl

        def start(k, buf, sem):
            off = pl.multiple_of(k * SH_W, TILE)
            pltpu.async_copy(lp_hbm.at[row, :, pl.ds(off, SH_W)], buf, sem)

        def waitfor(k, buf, sem):
            off = pl.multiple_of(k * SH_W, TILE)
            pltpu.make_async_copy(
                lp_hbm.at[row, :, pl.ds(off, SH_W)], buf, sem).wait()

        def proc(buf, k, cur):
            vbase = k * SH_W
            return lax.fori_loop(
                0, NBEAMS, mk_proc(buf, NGROUPS, GROUP, vbase, rl), cur)

        start(0, buf0, sem0)
        init = (_neg_inf_vec(), jnp.zeros((LANES,), jnp.int32))

        def it(i, cur):
            s = 2 * i
            start(s + 1, buf1, sem1)
            waitfor(s, buf0, sem0)
            cur = proc(buf0, s, cur)

            @pl.when(s + 2 < NSH)
            def _():
                start(s + 2, buf0, sem0)

            @pl.when(s + 2 >= NSH)
            def _():
                pltpu.async_copy(
                    lp_hbm.at[row, :, pl.ds(T13_BASE, T13_W)], buf13, sem13)

            waitfor(s + 1, buf1, sem1)
            cur = proc(buf1, s + 1, cur)
            return cur

        cur = lax.fori_loop(0, NSH // 2, it, init)

        pltpu.make_async_copy(
            lp_hbm.at[row, :, pl.ds(T13_BASE, T13_W)], buf13, sem13).wait()
        cur = lax.fori_loop(
            0, NBEAMS, mk_proc(buf13, NG13, G13, T13_BASE, rl), cur)

        pltpu.sync_copy(
            tail_hbm.at[pl.ds(row * (NBEAMS * VOCAB_TAIL),
                              NBEAMS * VOCAB_TAIL)], tbuf)

        def tail_fn(m, cc):
            cv2, ci2 = cc
            b = m // 2
            v = tbuf[pl.ds(m * LANES, LANES)]
            bsel = jnp.full((LANES,), rl * NBEAMS, jnp.int32) + b
            bias_b = jnp.take_along_axis(bias_all, bsel, axis=0)
            ts = t_safe_of(cv2, bias_b)
            h = jnp.any(v > ts)

            def do(cc2):
                cv3, ci3 = cc2
                bv = v + bias_b
                fi = b * VOCAB + VOCAB_MAIN + (m % 2) * LANES + lane
                return merge_vec(cv3, ci3, bv, fi)

            return lax.cond(h, do, lambda x: x, (cv2, ci2))

        cv, ci = lax.fori_loop(0, NBEAMS * 2, tail_fn, cur)

        vd = lax.rev(cv, (0,))
        idd = lax.rev(ci, (0,))
        stv[...] = vd
        sti[...] = idd % VOCAB
        stb[...] = idd // VOCAB
        pltpu.sync_copy(stv, out_s.at[row])
        pltpu.sync_copy(sti, out_i.at[row])
        pltpu.sync_copy(stb, out_b.at[row])
        return 0

    lax.fori_loop(0, ROWS_PER_W, row_fn, 0)


_topk_call = functools.partial(
    pl.kernel,
    out_type=(
        jax.ShapeDtypeStruct((BSZ, K), jnp.float32),
        jax.ShapeDtypeStruct((BSZ, K), jnp.int32),
        jax.ShapeDtypeStruct((BSZ, K), jnp.int32),
    ),
    mesh=plsc.VectorSubcoreMesh(core_axis_name="c", subcore_axis_name="s"),
    scratch_types=[
        pltpu.VMEM((NBEAMS, SH_W), jnp.float32),
        pltpu.VMEM((NBEAMS, SH_W), jnp.float32),
        pltpu.VMEM((NBEAMS, T13_W), jnp.float32),
        pltpu.VMEM((NBEAMS * VOCAB_TAIL,), jnp.float32),
        pltpu.VMEM((LANES,), jnp.float32),
        pltpu.VMEM((LANES,), jnp.float32),
        pltpu.VMEM((LANES,), jnp.int32),
        pltpu.VMEM((LANES,), jnp.int32),
        pltpu.SemaphoreType.DMA,
        pltpu.SemaphoreType.DMA,
        pltpu.SemaphoreType.DMA,
    ],
    compiler_params=pltpu.CompilerParams(needs_layout_passes=False),
)(_topk_body)


def kernel(step, lprobs, scores):
    step = jnp.asarray(step)
    col = lax.dynamic_index_in_dim(
        scores, jnp.maximum(step - 1, 0), axis=2, keepdims=False)
    step0_bias = jnp.where(jnp.arange(NBEAMS) == 0, 0.0, NEG_INF)[None, :]
    bias = jnp.where(step == 0, step0_bias, col).astype(jnp.float32)
    tail = lprobs[:, :, VOCAB_MAIN:].reshape(-1)
    return _topk_call(lprobs, tail, bias.reshape(-1))


# speedup vs baseline: 19.6126x; 1.0004x over previous
import functools

import jax
import jax.numpy as jnp
from jax import lax
from jax.experimental import pallas as pl
from jax.experimental.pallas import tpu as pltpu
from jax.experimental.pallas import tpu_sc as plsc

BSZ = 64
NBEAMS = 8
VOCAB = 100000
K = 16
LANES = 16

TILE = 128
NTILES_FULL = VOCAB // TILE
VOCAB_MAIN = NTILES_FULL * TILE
VOCAB_TAIL = VOCAB - VOCAB_MAIN

SH_TILES = 48
SH_W = SH_TILES * TILE
NSH = 16
T13_TILES = NTILES_FULL - NSH * SH_TILES
T13_W = T13_TILES * TILE
T13_BASE = NSH * SH_W

GROUP = 48
NGROUPS = (SH_W // LANES) // GROUP
G13 = 52
NG13 = (T13_W // LANES) // G13

NWORKERS = 32
ROWS_PER_W = BSZ // NWORKERS

NEG_INF = float("-inf")


def _neg_inf_vec():
    return jnp.full((LANES,), NEG_INF, jnp.float32)


def _tree_max(vs):
    vs = list(vs)
    while len(vs) > 1:
        vs = [jnp.maximum(a, b) for a, b in zip(vs[::2], vs[1::2])] + (
            [vs[-1]] if len(vs) % 2 else [])
    return vs[0]


def _topk_body(lp_hbm, tail_hbm, bias_hbm, out_s, out_i, out_b,
               buf0, buf1, buf13, tbuf, biasv, stv, sti, stb,
               sem0, sem1, sem13):
    cid = lax.axis_index("c")
    sid = lax.axis_index("s")
    wid = sid * 2 + cid

    pltpu.sync_copy(bias_hbm.at[pl.ds(wid * LANES, LANES)], biasv)
    bias_all = biasv[...]
    lane = lax.broadcasted_iota(jnp.int32, (LANES,), 0)
    zeros_idx = jnp.zeros((LANES,), jnp.int32)

    def t_safe_of(cur_val, bias_b):
        t = jnp.take_along_axis(cur_val, zeros_idx, axis=0)
        ta = t - bias_b
        margin = (jnp.abs(t) + jnp.abs(bias_b) + 8.0) * 1e-6
        return ta - margin

    def merge_vec(cv, ci, bv, fi):
        sk, si = plsc.sort_key_val(bv, fi)
        rk = lax.rev(sk, (0,))
        ri = lax.rev(si, (0,))
        ge = cv >= rk
        hv = jnp.where(ge, cv, rk)
        hi = jnp.where(ge, ci, ri)
        nv, ni = plsc.sort_key_val(hv, hi)
        return nv, ni

    pos_inf_vec = jnp.full((LANES,), float("inf"), jnp.float32)

    def mk_proc(buf, ngroups, group, vbase, rl):
        def beam_fn(b, cur):
            cv0, ci0 = cur
            bsel = jnp.full((LANES,), rl * NBEAMS, jnp.int32) + b
            bias_b = jnp.take_along_axis(bias_all, bsel, axis=0)
            fbase = b * VOCAB + vbase

            def grp(g, carry):
                cv, ci, ts = carry
                base = g * (group * LANES)
                vs = [buf[b, pl.ds(base + LANES * j, LANES)]
                      for j in range(group)]
                gmax = _tree_max(vs)
                hit = jnp.any(gmax > ts)

                def slow(c):
                    cv_in, ci_in, _ = c

                    def sweep(ceil_v, ceil_j):
                        gm = _neg_inf_vec()
                        gj = jnp.zeros((LANES,), jnp.int32)
                        for j in range(group):
                            bv = buf[b, pl.ds(base + LANES * j, LANES)] + bias_b
                            jj = jnp.full((LANES,), j, jnp.int32)
                            elig = (bv < ceil_v) | (
                                (bv == ceil_v) & (jj > ceil_j))
                            c2 = elig & (bv > gm)
                            gm = jnp.where(c2, bv, gm)
                            gj = jnp.where(c2, jj, gj)
                        return gm, gj

                    gm0, gj0 = sweep(pos_inf_vec,
                                     jnp.full((LANES,), -1, jnp.int32))
                    t0 = jnp.take_along_axis(cv_in, zeros_idx, axis=0)
                    hit0 = jnp.any(gm0 > t0)

                    def w_cond(st):
                        return st[4]

                    def w_body(st):
                        cv2, ci2, gm, gj, _ = st
                        fi = fbase + base + gj * LANES + lane
                        cv2, ci2 = merge_vec(cv2, ci2, gm, fi)
                        gm2, gj2 = sweep(gm, gj)
                        t_ex = jnp.take_along_axis(cv2, zeros_idx, axis=0)
                        return (cv2, ci2, gm2, gj2, jnp.any(gm2 > t_ex))

                    cv_o, ci_o, _, _, _ = lax.while_loop(
                        w_cond, w_body, (cv_in, ci_in, gm0, gj0, hit0))
                    return cv_o, ci_o, t_safe_of(cv_o, bias_b)

                return lax.cond(hit, slow, lambda c: c, (cv, ci, ts))

            cv1, ci1, _ = plsc.parallel_loop(
                0, ngroups, carry=(cv0, ci0, t_safe_of(cv0, bias_b)))(grp)
            return cv1, ci1

        return beam_fn

    def row_fn(rl, _):
        row = wid * ROWS_PER_W + rl

        def start(k, buf, sem):
            off = pl.multiple_of(k * SH_W, TILE)
            pltpu.async_copy(lp_hbm.at[row, :, pl.ds(off, SH_W)], buf, sem)

        def waitfor(k, buf, sem):
            off = pl.multiple_of(k * SH_W, TILE)
            pltpu.make_async_copy(
                lp_hbm.at[row, :, pl.ds(off, SH_W)], buf, sem).wait()

        def proc(buf, k, cur):
            vbase = k * SH_W
            return lax.fori_loop(
                0, NBEAMS, mk_proc(buf, NGROUPS, GROUP, vbase, rl), cur)

        start(0, buf0, sem0)
        init = (_neg_inf_vec(), jnp.zeros((LANES,), jnp.int32))

        def it(i, cur):
            s = 2 * i
            start(s + 1, buf1, sem1)
            waitfor(s, buf0, sem0)
            cur = proc(buf0, s, cur)

            @pl.when(s + 2 < NSH)
            def _():
                start(s + 2, buf0, sem0)

            @pl.when(s + 2 >= NSH)
            def _():
                pltpu.async_copy(
                    lp_hbm.at[row, :, pl.ds(T13_BASE, T13_W)], buf13, sem13)

            waitfor(s + 1, buf1, sem1)
            cur = proc(buf1, s + 1, cur)
            return cur

        cur = lax.fori_loop(0, NSH // 2, it, init)

        pltpu.make_async_copy(
            lp_hbm.at[row, :, pl.ds(T13_BASE, T13_W)], buf13, sem13).wait()
        cur = lax.fori_loop(
            0, NBEAMS, mk_proc(buf13, NG13, G13, T13_BASE, rl), cur)

        pltpu.sync_copy(
            tail_hbm.at[pl.ds(row * (NBEAMS * VOCAB_TAIL),
                              NBEAMS * VOCAB_TAIL)], tbuf)

        def tail_fn(m, cc):
            cv2, ci2 = cc
            b = m // 2
            v = tbuf[pl.ds(m * LANES, LANES)]
            bsel = jnp.full((LANES,), rl * NBEAMS, jnp.int32) + b
            bias_b = jnp.take_along_axis(bias_all, bsel, axis=0)
            ts = t_safe_of(cv2, bias_b)
            h = jnp.any(v > ts)

            def do(cc2):
                cv3, ci3 = cc2
                bv = v + bias_b
                fi = b * VOCAB + VOCAB_MAIN + (m % 2) * LANES + lane
                return merge_vec(cv3, ci3, bv, fi)

            return lax.cond(h, do, lambda x: x, (cv2, ci2))

        cv, ci = lax.fori_loop(0, NBEAMS * 2, tail_fn, cur)

        vd = lax.rev(cv, (0,))
        idd = lax.rev(ci, (0,))
        stv[...] = vd
        sti[...] = idd % VOCAB
        stb[...] = idd // VOCAB
        pltpu.sync_copy(stv, out_s.at[row])
        pltpu.sync_copy(sti, out_i.at[row])
        pltpu.sync_copy(stb, out_b.at[row])
        return 0

    lax.fori_loop(0, ROWS_PER_W, row_fn, 0)


_topk_call = functools.partial(
    pl.kernel,
    out_type=(
        jax.ShapeDtypeStruct((BSZ, K), jnp.float32),
        jax.ShapeDtypeStruct((BSZ, K), jnp.int32),
        jax.ShapeDtypeStruct((BSZ, K), jnp.int32),
    ),
    mesh=plsc.VectorSubcoreMesh(core_axis_name="c", subcore_axis_name="s"),
    scratch_types=[
        pltpu.VMEM((NBEAMS, SH_W), jnp.float32),
        pltpu.VMEM((NBEAMS, SH_W), jnp.float32),
        pltpu.VMEM((NBEAMS, T13_W), jnp.float32),
        pltpu.VMEM((NBEAMS * VOCAB_TAIL,), jnp.float32),
        pltpu.VMEM((LANES,), jnp.float32),
        pltpu.VMEM((LANES,), jnp.float32),
        pltpu.VMEM((LANES,), jnp.int32),
        pltpu.VMEM((LANES,), jnp.int32),
        pltpu.SemaphoreType.DMA,
        pltpu.SemaphoreType.DMA,
        pltpu.SemaphoreType.DMA,
    ],
    compiler_params=pltpu.CompilerParams(needs_layout_passes=False),
)(_topk_body)


def kernel(step, lprobs, scores):
    step = jnp.asarray(step)
    col = lax.dynamic_index_in_dim(
        scores, jnp.maximum(step - 1, 0), axis=2, keepdims=False)
    step0_bias = jnp.where(jnp.arange(NBEAMS) == 0, 0.0, NEG_INF)[None, :]
    bias = jnp.where(step == 0, step0_bias, col).astype(jnp.float32)
    tail = lprobs[:, :, VOCAB_MAIN:].reshape(-1)
    return _topk_call(lprobs, tail, bias.reshape(-1))


# speedup vs baseline: 24.0682x; 1.2272x over previous
import functools

import jax
import jax.numpy as jnp
from jax import lax
from jax.experimental import pallas as pl
from jax.experimental.pallas import tpu as pltpu
from jax.experimental.pallas import tpu_sc as plsc

BSZ = 64
NBEAMS = 8
VOCAB = 100000
K = 16
LANES = 16

TILE = 128
NTILES_FULL = VOCAB // TILE
VOCAB_MAIN = NTILES_FULL * TILE
VOCAB_TAIL = VOCAB - VOCAB_MAIN

SH_TILES = 48
SH_W = SH_TILES * TILE
NSH = 16
T13_TILES = NTILES_FULL - NSH * SH_TILES
T13_W = T13_TILES * TILE
T13_BASE = NSH * SH_W

GROUP = 48
NGROUPS = (SH_W // LANES) // GROUP
G13 = 52
NG13 = (T13_W // LANES) // G13

NWORKERS = 32
ROWS_PER_W = BSZ // NWORKERS

NEG_INF = float("-inf")


def _neg_inf_vec():
    return jnp.full((LANES,), NEG_INF, jnp.float32)


def _tree_max(vs):
    vs = list(vs)
    while len(vs) > 1:
        vs = [jnp.maximum(a, b) for a, b in zip(vs[::2], vs[1::2])] + (
            [vs[-1]] if len(vs) % 2 else [])
    return vs[0]


def _topk_body(lp_hbm, tail_hbm, bias_hbm, out_s, out_i, out_b,
               buf0, buf1, buf13, tbuf, gbuf, biasv, stv, sti, stb,
               sem0, sem1, sem13):
    cid = lax.axis_index("c")
    sid = lax.axis_index("s")
    wid = sid * 2 + cid

    pltpu.sync_copy(bias_hbm.at[pl.ds(wid * LANES, LANES)], biasv)
    bias_all = biasv[...]
    lane = lax.broadcasted_iota(jnp.int32, (LANES,), 0)
    zeros_idx = jnp.zeros((LANES,), jnp.int32)

    def t_safe_of(cur_val, bias_b):
        t = jnp.take_along_axis(cur_val, zeros_idx, axis=0)
        ta = t - bias_b
        margin = (jnp.abs(t) + jnp.abs(bias_b) + 8.0) * 1e-6
        return ta - margin

    def merge_vec(cv, ci, bv, fi):
        sk, si = plsc.sort_key_val(bv, fi)
        rk = lax.rev(sk, (0,))
        ri = lax.rev(si, (0,))
        ge = cv >= rk
        hv = jnp.where(ge, cv, rk)
        hi = jnp.where(ge, ci, ri)
        nv, ni = plsc.sort_key_val(hv, hi)
        return nv, ni

    pos_inf_vec = jnp.full((LANES,), float("inf"), jnp.float32)

    def mk_hot(buf, ngroups, group):
        def beam_hot(b, c):
            def grp(g):
                base = g * (group * LANES)
                vs = [buf[b, pl.ds(base + LANES * j, LANES)]
                      for j in range(group)]
                gbuf[pl.ds((b * ngroups + g) * LANES, LANES)] = _tree_max(vs)

            plsc.parallel_loop(0, ngroups)(grp)
            return c

        return beam_hot

    def mk_deferred(buf, ngroups, group, vbase, rl):
        def beam_fn(b, cur):
            cv0, ci0 = cur
            bsel = jnp.full((LANES,), rl * NBEAMS, jnp.int32) + b
            bias_b = jnp.take_along_axis(bias_all, bsel, axis=0)
            fbase = b * VOCAB + vbase
            ts0 = t_safe_of(cv0, bias_b)
            gvs = [gbuf[pl.ds((b * ngroups + g) * LANES, LANES)]
                   for g in range(ngroups)]
            bhit = jnp.any(_tree_max(gvs) > ts0)

            def slow_beam(c0):
                def g_loop(g, cc):
                    cv, ci, ts = cc
                    gv = gbuf[pl.ds((b * ngroups + g) * LANES, LANES)]
                    ghit = jnp.any(gv > ts)
                    base = g * (group * LANES)

                    def slow(c):
                        cv_in, ci_in, _ = c

                        def sweep(ceil_v, ceil_j):
                            gm = _neg_inf_vec()
                            gj = jnp.zeros((LANES,), jnp.int32)
                            for j in range(group):
                                bv = (buf[b, pl.ds(base + LANES * j, LANES)]
                                      + bias_b)
                                jj = jnp.full((LANES,), j, jnp.int32)
                                elig = (bv < ceil_v) | (
                                    (bv == ceil_v) & (jj > ceil_j))
                                c2 = elig & (bv > gm)
                                gm = jnp.where(c2, bv, gm)
                                gj = jnp.where(c2, jj, gj)
                            return gm, gj

                        gm0, gj0 = sweep(pos_inf_vec,
                                         jnp.full((LANES,), -1, jnp.int32))
                        t0 = jnp.take_along_axis(cv_in, zeros_idx, axis=0)
                        hit0 = jnp.any(gm0 > t0)

                        def w_cond(st):
                            return st[4]

                        def w_body(st):
                            cv2, ci2, gm, gj, _ = st
                            fi = fbase + base + gj * LANES + lane
                            cv2, ci2 = merge_vec(cv2, ci2, gm, fi)
                            gm2, gj2 = sweep(gm, gj)
                            t_ex = jnp.take_along_axis(cv2, zeros_idx, axis=0)
                            return (cv2, ci2, gm2, gj2,
                                    jnp.any(gm2 > t_ex))

                        cv_o, ci_o, _, _, _ = lax.while_loop(
                            w_cond, w_body, (cv_in, ci_in, gm0, gj0, hit0))
                        return cv_o, ci_o, t_safe_of(cv_o, bias_b)

                    return lax.cond(ghit, slow, lambda c: c, (cv, ci, ts))

                return lax.fori_loop(0, ngroups, g_loop, c0)

            cv1, ci1, _ = lax.cond(
                bhit, slow_beam, lambda c: c, (cv0, ci0, ts0))
            return cv1, ci1

        return beam_fn

    def row_fn(rl, _):
        row = wid * ROWS_PER_W + rl

        def start(k, buf, sem):
            off = pl.multiple_of(k * SH_W, TILE)
            pltpu.async_copy(lp_hbm.at[row, :, pl.ds(off, SH_W)], buf, sem)

        def waitfor(k, buf, sem):
            off = pl.multiple_of(k * SH_W, TILE)
            pltpu.make_async_copy(
                lp_hbm.at[row, :, pl.ds(off, SH_W)], buf, sem).wait()

        def proc(buf, k, cur):
            vbase = k * SH_W
            lax.fori_loop(0, NBEAMS, mk_hot(buf, NGROUPS, GROUP), 0)
            return lax.fori_loop(
                0, NBEAMS, mk_deferred(buf, NGROUPS, GROUP, vbase, rl), cur)

        start(0, buf0, sem0)
        init = (_neg_inf_vec(), jnp.zeros((LANES,), jnp.int32))

        def it(i, cur):
            s = 2 * i
            start(s + 1, buf1, sem1)
            waitfor(s, buf0, sem0)
            cur = proc(buf0, s, cur)

            @pl.when(s + 2 < NSH)
            def _():
                start(s + 2, buf0, sem0)

            @pl.when(s + 2 >= NSH)
            def _():
                pltpu.async_copy(
                    lp_hbm.at[row, :, pl.ds(T13_BASE, T13_W)], buf13, sem13)

            waitfor(s + 1, buf1, sem1)
            cur = proc(buf1, s + 1, cur)
            return cur

        cur = lax.fori_loop(0, NSH // 2, it, init)

        pltpu.make_async_copy(
            lp_hbm.at[row, :, pl.ds(T13_BASE, T13_W)], buf13, sem13).wait()
        lax.fori_loop(0, NBEAMS, mk_hot(buf13, NG13, G13), 0)
        cur = lax.fori_loop(
            0, NBEAMS, mk_deferred(buf13, NG13, G13, T13_BASE, rl), cur)

        pltpu.sync_copy(
            tail_hbm.at[pl.ds(row * (NBEAMS * VOCAB_TAIL),
                              NBEAMS * VOCAB_TAIL)], tbuf)

        def tail_fn(m, cc):
            cv2, ci2 = cc
            b = m // 2
            v = tbuf[pl.ds(m * LANES, LANES)]
            bsel = jnp.full((LANES,), rl * NBEAMS, jnp.int32) + b
            bias_b = jnp.take_along_axis(bias_all, bsel, axis=0)
            ts = t_safe_of(cv2, bias_b)
            h = jnp.any(v > ts)

            def do(cc2):
                cv3, ci3 = cc2
                bv = v + bias_b
                fi = b * VOCAB + VOCAB_MAIN + (m % 2) * LANES + lane
                return merge_vec(cv3, ci3, bv, fi)

            return lax.cond(h, do, lambda x: x, (cv2, ci2))

        cv, ci = lax.fori_loop(0, NBEAMS * 2, tail_fn, cur)

        vd = lax.rev(cv, (0,))
        idd = lax.rev(ci, (0,))
        stv[...] = vd
        sti[...] = idd % VOCAB
        stb[...] = idd // VOCAB
        pltpu.sync_copy(stv, out_s.at[row])
        pltpu.sync_copy(sti, out_i.at[row])
        pltpu.sync_copy(stb, out_b.at[row])
        return 0

    lax.fori_loop(0, ROWS_PER_W, row_fn, 0)


_topk_call = functools.partial(
    pl.kernel,
    out_type=(
        jax.ShapeDtypeStruct((BSZ, K), jnp.float32),
        jax.ShapeDtypeStruct((BSZ, K), jnp.int32),
        jax.ShapeDtypeStruct((BSZ, K), jnp.int32),
    ),
    mesh=plsc.VectorSubcoreMesh(core_axis_name="c", subcore_axis_name="s"),
    scratch_types=[
        pltpu.VMEM((NBEAMS, SH_W), jnp.float32),
        pltpu.VMEM((NBEAMS, SH_W), jnp.float32),
        pltpu.VMEM((NBEAMS, T13_W), jnp.float32),
        pltpu.VMEM((NBEAMS * VOCAB_TAIL,), jnp.float32),
        pltpu.VMEM((NBEAMS * NGROUPS * LANES,), jnp.float32),
        pltpu.VMEM((LANES,), jnp.float32),
        pltpu.VMEM((LANES,), jnp.float32),
        pltpu.VMEM((LANES,), jnp.int32),
        pltpu.VMEM((LANES,), jnp.int32),
        pltpu.SemaphoreType.DMA,
        pltpu.SemaphoreType.DMA,
        pltpu.SemaphoreType.DMA,
    ],
    compiler_params=pltpu.CompilerParams(needs_layout_passes=False),
)(_topk_body)


def kernel(step, lprobs, scores):
    step = jnp.asarray(step)
    col = lax.dynamic_index_in_dim(
        scores, jnp.maximum(step - 1, 0), axis=2, keepdims=False)
    step0_bias = jnp.where(jnp.arange(NBEAMS) == 0, 0.0, NEG_INF)[None, :]
    bias = jnp.where(step == 0, step0_bias, col).astype(jnp.float32)
    tail = lprobs[:, :, VOCAB_MAIN:].reshape(-1)
    return _topk_call(lprobs, tail, bias.reshape(-1))
